# Initial kernel scaffold; baseline (speedup 1.0000x reference)
import jax
import jax.numpy as jnp
from jax.experimental import pallas as pl


def kernel(states, model_ids, arcs_weights, ilabels, to_states, start_end_arcs, backoff_to_states, backoff_weights, final_weights, model2alpha):
    raise NotImplementedError("write your pallas kernel here")


# speedup vs baseline: 6.7896x; 6.7896x over previous
import functools

import jax
import jax.numpy as jnp
from jax import lax
from jax.experimental import pallas as pl
from jax.experimental.pallas import tpu as pltpu
from jax.experimental.pallas import tpu_sc as plsc

VOCAB = 1024
B = 256
MAX_ARCS = 64
MAX_ORDER = 3
NUM_WORKERS = 32
ROWS = B // NUM_WORKERS
L = 16


def _advance_body(states_h, alpha_h, fwb_h, cur3_h, starts_h, ends_h, bwl_h,
                  aw_h, il_h, ts_h, out_s_h, out_n_h,
                  st_v, alpha_v, fw_v, cur3_v, ab3_v,
                  ss0_v, ss1_v, ss2_v, ee0_v, ee1_v, ee2_v,
                  bw0_v, bw1_v, bw2_v,
                  idx_v, lab_v, w_v, to_v, bl_v, s_v, n_v,
                  sem_pre, sem_arc):
    cc_ = lax.axis_index("c")
    ss_ = lax.axis_index("s")
    wid = ss_ * 2 + cc_
    base = wid * ROWS

    iota = lax.iota(jnp.int32, L)
    lane_lo = iota < ROWS
    r8 = jnp.bitwise_and(iota, ROWS - 1)
    three16 = jnp.full((L,), MAX_ORDER, jnp.int32)

    descs = []
    for src, dst in ((states_h, st_v), (alpha_h, alpha_v), (fwb_h, fw_v),
                     (cur3_h, cur3_v)):
        descs.append(pltpu.async_copy(src.at[pl.ds(base, ROWS)],
                                      dst.at[pl.ds(0, ROWS)], sem_pre))
    lvl_refs = ((ss0_v, ee0_v, bw0_v), (ss1_v, ee1_v, bw1_v),
                (ss2_v, ee2_v, bw2_v))
    for lvl in range(MAX_ORDER):
        off = lvl * B + base
        for src, dst in zip((starts_h, ends_h, bwl_h), lvl_refs[lvl]):
            descs.append(pltpu.async_copy(src.at[pl.ds(off, ROWS)],
                                          dst.at[pl.ds(0, ROWS)], sem_pre))
    for d in descs:
        d.wait()

    for r in range(ROWS):
        def init_body(i, carry, r=r):
            bl_v[r, pl.ds(i * L, L)] = three16
            return carry
        lax.fori_loop(0, VOCAB // L, init_body, 0)

    ab = jnp.zeros((L,), jnp.float32)
    for lvl in range(MAX_ORDER):
        ss_v, ee_v, bw_v = lvl_refs[lvl]
        starts = plsc.load_gather(ss_v, [r8])
        ends = plsc.load_gather(ee_v, [r8])
        lens = ends - starts

        for m in range(MAX_ARCS):
            idx_v[m // 8, pl.ds((m % 8) * L, L)] = starts + m
        descs = []
        for c4 in range(MAX_ARCS * L // 128):
            dst = pl.ds(128 * c4, 128)
            descs.append(pltpu.async_copy(il_h.at[idx_v.at[c4]],
                                          lab_v.at[dst], sem_arc))
            descs.append(pltpu.async_copy(aw_h.at[idx_v.at[c4]],
                                          w_v.at[dst], sem_arc))
            descs.append(pltpu.async_copy(ts_h.at[idx_v.at[c4]],
                                          to_v.at[dst], sem_arc))
        for d in descs:
            d.wait()

        lvec = jnp.full((L,), lvl, jnp.int32)
        abl = ab

        def arc_body(j, carry, lvec=lvec, abl=abl, lens=lens):
            off = j * L
            lab = jnp.bitwise_and(lab_v[pl.ds(off, L)], VOCAB - 1)
            wv = w_v[pl.ds(off, L)]
            tv = to_v[pl.ds(off, L)]
            valid = lane_lo & (j < lens)
            cur_bl = plsc.load_gather(bl_v, [r8, lab], mask=valid)
            cur_s = plsc.load_gather(s_v, [r8, lab], mask=valid)
            cur_n = plsc.load_gather(n_v, [r8, lab], mask=valid)
            cand = abl + wv
            better = lvec < cur_bl
            eq = lvec == cur_bl
            new_s = jnp.where(better, cand,
                              jnp.where(eq, jnp.maximum(cur_s, cand), cur_s))
            new_n = jnp.where(better, tv,
                              jnp.where(eq, jnp.maximum(cur_n, tv), cur_n))
            new_bl = jnp.minimum(cur_bl, lvec)
            plsc.store_scatter(s_v, [r8, lab], new_s, mask=valid)
            plsc.store_scatter(n_v, [r8, lab], new_n, mask=valid)
            plsc.store_scatter(bl_v, [r8, lab], new_bl, mask=valid)
            return carry

        lax.fori_loop(0, MAX_ARCS, arc_body, 0)
        ab = ab + bw_v[...]

    ab3_v[...] = ab

    def fin_body(i, carry):
        r = lax.shift_right_logical(i, 6)
        ci = jnp.bitwise_and(i, VOCAB // L - 1)
        rv = jnp.full((L,), r, jnp.int32)
        col = ci * L + iota
        ab3r = plsc.load_gather(ab3_v, [rv])
        cur3r = plsc.load_gather(cur3_v, [rv])
        alphar = plsc.load_gather(alpha_v, [rv])
        fwr = plsc.load_gather(fw_v, [rv])
        str_ = plsc.load_gather(st_v, [rv])
        blv = plsc.load_gather(bl_v, [rv, col])
        svv = plsc.load_gather(s_v, [rv, col])
        nvv = plsc.load_gather(n_v, [rv, col])
        found = blv < three16
        outs = jnp.where(found, svv, ab3r)
        outn = jnp.where(found, nvv, cur3r)
        eos = col == 0
        outs = jnp.where(eos, fwr, outs) * alphar
        outn = jnp.where(eos, str_, outn)
        plsc.store_scatter(s_v, [rv, col], outs)
        plsc.store_scatter(n_v, [rv, col], outn)
        return carry

    lax.fori_loop(0, ROWS * (VOCAB // L), fin_body, 0)

    pltpu.sync_copy(s_v, out_s_h.at[pl.ds(base, ROWS)])
    pltpu.sync_copy(n_v, out_n_h.at[pl.ds(base, ROWS)])


_advance = functools.partial(
    pl.kernel,
    out_type=(jax.ShapeDtypeStruct((B, VOCAB), jnp.float32),
              jax.ShapeDtypeStruct((B, VOCAB), jnp.int32)),
    mesh=plsc.VectorSubcoreMesh(core_axis_name="c", subcore_axis_name="s"),
    compiler_params=pltpu.CompilerParams(needs_layout_passes=False),
    scratch_types=[
        pltpu.VMEM((L,), jnp.int32),
        pltpu.VMEM((L,), jnp.float32),
        pltpu.VMEM((L,), jnp.float32),
        pltpu.VMEM((L,), jnp.int32),
        pltpu.VMEM((L,), jnp.float32),
        pltpu.VMEM((L,), jnp.int32),
        pltpu.VMEM((L,), jnp.int32),
        pltpu.VMEM((L,), jnp.int32),
        pltpu.VMEM((L,), jnp.int32),
        pltpu.VMEM((L,), jnp.int32),
        pltpu.VMEM((L,), jnp.int32),
        pltpu.VMEM((L,), jnp.float32),
        pltpu.VMEM((L,), jnp.float32),
        pltpu.VMEM((L,), jnp.float32),
        pltpu.VMEM((8, 128), jnp.int32),
        pltpu.VMEM((1024,), jnp.int32),
        pltpu.VMEM((1024,), jnp.float32),
        pltpu.VMEM((1024,), jnp.int32),
        pltpu.VMEM((ROWS, VOCAB), jnp.int32),
        pltpu.VMEM((ROWS, VOCAB), jnp.float32),
        pltpu.VMEM((ROWS, VOCAB), jnp.int32),
        pltpu.SemaphoreType.DMA,
        pltpu.SemaphoreType.DMA,
    ],
)(_advance_body)


def kernel(states, model_ids, arcs_weights, ilabels, to_states,
           start_end_arcs, backoff_to_states, backoff_weights,
           final_weights, model2alpha):
    cur0 = states
    cur1 = backoff_to_states[cur0]
    cur2 = backoff_to_states[cur1]
    cur3 = backoff_to_states[cur2]
    curcat = jnp.concatenate([cur0, cur1, cur2])
    se = start_end_arcs[curcat]
    starts = se[:, 0]
    ends = se[:, 1]
    bwl = backoff_weights[curcat]
    fwb = final_weights[states]
    alpha = model2alpha[model_ids]
    return _advance(states, alpha, fwb, cur3, starts, ends, bwl,
                    arcs_weights, ilabels, to_states)


# speedup vs baseline: 7.6676x; 1.1293x over previous
import functools

import jax
import jax.numpy as jnp
from jax import lax
from jax.experimental import pallas as pl
from jax.experimental.pallas import tpu as pltpu
from jax.experimental.pallas import tpu_sc as plsc

VOCAB = 1024
B = 256
MAX_ARCS = 64
MAX_ORDER = 3
NUM_WORKERS = 32
ROWS = B // NUM_WORKERS
L = 16


def _advance_body(states_h, alpha_h, fwb_h, cur3_h, starts_h, ends_h, bwl_h,
                  aw_h, il_h, ts_h, out_s_h, out_n_h,
                  st_v, alpha_v, fw_v, cur3_v, sf_v,
                  ss0_v, ss1_v, ss2_v, ee0_v, ee1_v, ee2_v,
                  bw0_v, bw1_v, bw2_v, idx_v,
                  lab0_v, w0_v, to0_v, lab1_v, w1_v, to1_v,
                  lab2_v, w2_v, to2_v,
                  bl_v, s_v, n_v,
                  sem_pre, sem_a0, sem_a1, sem_a2):
    cc_ = lax.axis_index("c")
    ss_ = lax.axis_index("s")
    wid = ss_ * 2 + cc_
    base = wid * ROWS

    iota = lax.iota(jnp.int32, L)
    lane_lo = iota < ROWS
    r8 = jnp.bitwise_and(iota, ROWS - 1)
    three16 = jnp.full((L,), MAX_ORDER, jnp.int32)
    zeros16 = jnp.zeros((L,), jnp.int32)

    descs = []
    for src, dst in ((states_h, st_v), (alpha_h, alpha_v), (fwb_h, fw_v),
                     (cur3_h, cur3_v)):
        descs.append(pltpu.async_copy(src.at[pl.ds(base, ROWS)],
                                      dst.at[pl.ds(0, ROWS)], sem_pre))
    lvl_refs = ((ss0_v, ee0_v, bw0_v), (ss1_v, ee1_v, bw1_v),
                (ss2_v, ee2_v, bw2_v))
    for lvl in range(MAX_ORDER):
        off = lvl * B + base
        for src, dst in zip((starts_h, ends_h, bwl_h), lvl_refs[lvl]):
            descs.append(pltpu.async_copy(src.at[pl.ds(off, ROWS)],
                                          dst.at[pl.ds(0, ROWS)], sem_pre))
    for d in descs:
        d.wait()

    alphav = plsc.load_gather(alpha_v, [r8])
    bwv = [plsc.load_gather(bw_v, [r8]) for _, _, bw_v in lvl_refs]
    ab_lvl = [jnp.zeros((L,), jnp.float32)]
    for lvl in range(MAX_ORDER):
        ab_lvl.append(ab_lvl[lvl] + bwv[lvl])
    sfill = ab_lvl[MAX_ORDER] * alphav
    sf_v[...] = sfill

    startsv = []
    lensv = []
    for lvl in range(MAX_ORDER):
        ss_ref, ee_ref, _ = lvl_refs[lvl]
        stv = plsc.load_gather(ss_ref, [r8])
        env = plsc.load_gather(ee_ref, [r8])
        startsv.append(stv)
        lensv.append(env - stv)
        for m in range(MAX_ARCS):
            idx_v[lvl * 8 + m // 8, pl.ds((m % 8) * L, L)] = stv + m

    arc_bufs = ((lab0_v, w0_v, to0_v), (lab1_v, w1_v, to1_v),
                (lab2_v, w2_v, to2_v))
    arc_sems = (sem_a0, sem_a1, sem_a2)
    lvl_descs = []
    for lvl in range(MAX_ORDER):
        lab_v, w_v, to_v = arc_bufs[lvl]
        dd = []
        for c4 in range(MAX_ARCS * L // 128):
            dst = pl.ds(128 * c4, 128)
            src = idx_v.at[lvl * 8 + c4]
            dd.append(pltpu.async_copy(il_h.at[src], lab_v.at[dst],
                                       arc_sems[lvl]))
            dd.append(pltpu.async_copy(aw_h.at[src], w_v.at[dst],
                                       arc_sems[lvl]))
            dd.append(pltpu.async_copy(ts_h.at[src], to_v.at[dst],
                                       arc_sems[lvl]))
        lvl_descs.append(dd)

    def init_body(i, carry):
        r = lax.shift_right_logical(i, 6)
        rsp = jnp.full((L,), r, jnp.int32)
        colv = jnp.bitwise_and(i, 63) * L + iota
        sfr = plsc.load_gather(sf_v, [rsp])
        c3r = plsc.load_gather(cur3_v, [rsp])
        plsc.store_scatter(bl_v, [rsp, colv], three16)
        plsc.store_scatter(s_v, [rsp, colv], sfr)
        plsc.store_scatter(n_v, [rsp, colv], c3r)
        return carry

    lax.fori_loop(0, ROWS * (VOCAB // L), init_body, 0)

    for lvl in range(MAX_ORDER):
        for d in lvl_descs[lvl]:
            d.wait()
        lab_v, w_v, to_v = arc_bufs[lvl]
        lvec = jnp.full((L,), lvl, jnp.int32)
        abl = ab_lvl[lvl]
        lens = lensv[lvl]

        def arc_body(j, carry, lab_v=lab_v, w_v=w_v, to_v=to_v,
                     lvec=lvec, abl=abl, lens=lens):
            off = j * L
            lab = jnp.bitwise_and(lab_v[pl.ds(off, L)], VOCAB - 1)
            wv = w_v[pl.ds(off, L)]
            tv = to_v[pl.ds(off, L)]
            valid = lane_lo & (j < lens)
            cur_bl = plsc.load_gather(bl_v, [r8, lab], mask=valid)
            cur_s = plsc.load_gather(s_v, [r8, lab], mask=valid)
            cur_n = plsc.load_gather(n_v, [r8, lab], mask=valid)
            cand = (abl + wv) * alphav
            better = lvec < cur_bl
            eq = lvec == cur_bl
            new_s = jnp.where(better, cand,
                              jnp.where(eq, jnp.maximum(cur_s, cand), cur_s))
            new_n = jnp.where(better, tv,
                              jnp.where(eq, jnp.maximum(cur_n, tv), cur_n))
            new_bl = jnp.minimum(cur_bl, lvec)
            plsc.store_scatter(s_v, [r8, lab], new_s, mask=valid)
            plsc.store_scatter(n_v, [r8, lab], new_n, mask=valid)
            plsc.store_scatter(bl_v, [r8, lab], new_bl, mask=valid)
            return carry

        lax.fori_loop(0, MAX_ARCS, arc_body, 0)

    fwr = plsc.load_gather(fw_v, [r8])
    str_ = plsc.load_gather(st_v, [r8])
    plsc.store_scatter(s_v, [r8, zeros16], fwr, mask=lane_lo)
    plsc.store_scatter(n_v, [r8, zeros16], str_, mask=lane_lo)

    pltpu.sync_copy(s_v, out_s_h.at[pl.ds(base, ROWS)])
    pltpu.sync_copy(n_v, out_n_h.at[pl.ds(base, ROWS)])


_advance = functools.partial(
    pl.kernel,
    out_type=(jax.ShapeDtypeStruct((B, VOCAB), jnp.float32),
              jax.ShapeDtypeStruct((B, VOCAB), jnp.int32)),
    mesh=plsc.VectorSubcoreMesh(core_axis_name="c", subcore_axis_name="s"),
    compiler_params=pltpu.CompilerParams(needs_layout_passes=False),
    scratch_types=[
        pltpu.VMEM((L,), jnp.int32),
        pltpu.VMEM((L,), jnp.float32),
        pltpu.VMEM((L,), jnp.float32),
        pltpu.VMEM((L,), jnp.int32),
        pltpu.VMEM((L,), jnp.float32),
        pltpu.VMEM((L,), jnp.int32),
        pltpu.VMEM((L,), jnp.int32),
        pltpu.VMEM((L,), jnp.int32),
        pltpu.VMEM((L,), jnp.int32),
        pltpu.VMEM((L,), jnp.int32),
        pltpu.VMEM((L,), jnp.int32),
        pltpu.VMEM((L,), jnp.float32),
        pltpu.VMEM((L,), jnp.float32),
        pltpu.VMEM((L,), jnp.float32),
        pltpu.VMEM((24, 128), jnp.int32),
        pltpu.VMEM((1024,), jnp.int32),
        pltpu.VMEM((1024,), jnp.float32),
        pltpu.VMEM((1024,), jnp.int32),
        pltpu.VMEM((1024,), jnp.int32),
        pltpu.VMEM((1024,), jnp.float32),
        pltpu.VMEM((1024,), jnp.int32),
        pltpu.VMEM((1024,), jnp.int32),
        pltpu.VMEM((1024,), jnp.float32),
        pltpu.VMEM((1024,), jnp.int32),
        pltpu.VMEM((ROWS, VOCAB), jnp.int32),
        pltpu.VMEM((ROWS, VOCAB), jnp.float32),
        pltpu.VMEM((ROWS, VOCAB), jnp.int32),
        pltpu.SemaphoreType.DMA,
        pltpu.SemaphoreType.DMA,
        pltpu.SemaphoreType.DMA,
        pltpu.SemaphoreType.DMA,
    ],
)(_advance_body)


def kernel(states, model_ids, arcs_weights, ilabels, to_states,
           start_end_arcs, backoff_to_states, backoff_weights,
           final_weights, model2alpha):
    cur0 = states
    cur1 = backoff_to_states[cur0]
    cur2 = backoff_to_states[cur1]
    cur3 = backoff_to_states[cur2]
    curcat = jnp.concatenate([cur0, cur1, cur2])
    se = start_end_arcs[curcat]
    starts = se[:, 0]
    ends = se[:, 1]
    bwl = backoff_weights[curcat]
    alpha = model2alpha[model_ids]
    fwb = final_weights[states] * alpha
    return _advance(states, alpha, fwb, cur3, starts, ends, bwl,
                    arcs_weights, ilabels, to_states)


# speedup vs baseline: 7.6717x; 1.0005x over previous
import functools

import jax
import jax.numpy as jnp
from jax import lax
from jax.experimental import pallas as pl
from jax.experimental.pallas import tpu as pltpu
from jax.experimental.pallas import tpu_sc as plsc

VOCAB = 1024
B = 256
MAX_ARCS = 64
MAX_ORDER = 3
NUM_WORKERS = 32
ROWS = B // NUM_WORKERS
L = 16


def _advance_body(states_h, alpha_h, fwb_h, cur3_h, starts_h, ends_h, bwl_h,
                  aw_h, il_h, ts_h, out_s_h, out_n_h,
                  st_v, alpha_v, fw_v, cur3_v, sf_v,
                  ss0_v, ss1_v, ss2_v, ee0_v, ee1_v, ee2_v,
                  bw0_v, bw1_v, bw2_v, idx_v,
                  lab0_v, w0_v, to0_v, lab1_v, w1_v, to1_v,
                  lab2_v, w2_v, to2_v,
                  bl_v, s_v, n_v,
                  sem_pre, sem_a0, sem_a1, sem_a2):
    cc_ = lax.axis_index("c")
    ss_ = lax.axis_index("s")
    wid = ss_ * 2 + cc_
    base = wid * ROWS

    iota = lax.iota(jnp.int32, L)
    lane_lo = iota < ROWS
    r8 = jnp.bitwise_and(iota, ROWS - 1)
    three16 = jnp.full((L,), MAX_ORDER, jnp.int32)
    zeros16 = jnp.zeros((L,), jnp.int32)

    descs = []
    for src, dst in ((states_h, st_v), (alpha_h, alpha_v), (fwb_h, fw_v),
                     (cur3_h, cur3_v)):
        descs.append(pltpu.async_copy(src.at[pl.ds(base, ROWS)],
                                      dst.at[pl.ds(0, ROWS)], sem_pre))
    lvl_refs = ((ss0_v, ee0_v, bw0_v), (ss1_v, ee1_v, bw1_v),
                (ss2_v, ee2_v, bw2_v))
    for lvl in range(MAX_ORDER):
        off = lvl * B + base
        for src, dst in zip((starts_h, ends_h, bwl_h), lvl_refs[lvl]):
            descs.append(pltpu.async_copy(src.at[pl.ds(off, ROWS)],
                                          dst.at[pl.ds(0, ROWS)], sem_pre))
    for d in descs:
        d.wait()

    alphav = plsc.load_gather(alpha_v, [r8])
    bwv = [plsc.load_gather(bw_v, [r8]) for _, _, bw_v in lvl_refs]
    ab_lvl = [jnp.zeros((L,), jnp.float32)]
    for lvl in range(MAX_ORDER):
        ab_lvl.append(ab_lvl[lvl] + bwv[lvl])
    sfill = ab_lvl[MAX_ORDER] * alphav
    sf_v[...] = sfill

    startsv = []
    lensv = []
    for lvl in range(MAX_ORDER):
        ss_ref, ee_ref, _ = lvl_refs[lvl]
        stv = plsc.load_gather(ss_ref, [r8])
        env = plsc.load_gather(ee_ref, [r8])
        startsv.append(stv)
        lensv.append(env - stv)
        for m in range(MAX_ARCS):
            idx_v[lvl * 8 + m // 8, pl.ds((m % 8) * L, L)] = stv + m

    arc_bufs = ((lab0_v, w0_v, to0_v), (lab1_v, w1_v, to1_v),
                (lab2_v, w2_v, to2_v))
    arc_sems = (sem_a0, sem_a1, sem_a2)
    lvl_descs = []
    for lvl in range(MAX_ORDER):
        lab_v, w_v, to_v = arc_bufs[lvl]
        dd = []
        for c4 in range(MAX_ARCS * L // 128):
            dst = pl.ds(128 * c4, 128)
            src = idx_v.at[lvl * 8 + c4]
            dd.append(pltpu.async_copy(il_h.at[src], lab_v.at[dst],
                                       arc_sems[lvl]))
            dd.append(pltpu.async_copy(aw_h.at[src], w_v.at[dst],
                                       arc_sems[lvl]))
            dd.append(pltpu.async_copy(ts_h.at[src], to_v.at[dst],
                                       arc_sems[lvl]))
        lvl_descs.append(dd)

    def init_body(i, carry):
        r = lax.shift_right_logical(i, 6)
        rsp = jnp.full((L,), r, jnp.int32)
        colv = jnp.bitwise_and(i, 63) * L + iota
        sfr = plsc.load_gather(sf_v, [rsp])
        c3r = plsc.load_gather(cur3_v, [rsp])
        plsc.store_scatter(bl_v, [rsp, colv], three16)
        plsc.store_scatter(s_v, [rsp, colv], sfr)
        plsc.store_scatter(n_v, [rsp, colv], c3r)
        return carry

    lax.fori_loop(0, ROWS * (VOCAB // L), init_body, 0)

    for lvl in range(MAX_ORDER):
        for d in lvl_descs[lvl]:
            d.wait()
        lab_v, w_v, to_v = arc_bufs[lvl]
        lvec = jnp.full((L,), lvl, jnp.int32)
        abl = ab_lvl[lvl]
        lens = lensv[lvl]

        def arc_body(j, carry, lab_v=lab_v, w_v=w_v, to_v=to_v,
                     lvec=lvec, abl=abl, lens=lens):
            off = j * L
            lab = jnp.bitwise_and(lab_v[pl.ds(off, L)], VOCAB - 1)
            wv = w_v[pl.ds(off, L)]
            tv = to_v[pl.ds(off, L)]
            valid = lane_lo & (j < lens)
            cur_bl = plsc.load_gather(bl_v, [r8, lab], mask=valid)
            cur_s = plsc.load_gather(s_v, [r8, lab], mask=valid)
            cur_n = plsc.load_gather(n_v, [r8, lab], mask=valid)
            cand = (abl + wv) * alphav
            better = lvec < cur_bl
            eq = lvec == cur_bl
            new_s = jnp.where(better, cand,
                              jnp.where(eq, jnp.maximum(cur_s, cand), cur_s))
            new_n = jnp.where(better, tv,
                              jnp.where(eq, jnp.maximum(cur_n, tv), cur_n))
            new_bl = jnp.minimum(cur_bl, lvec)
            plsc.store_scatter(s_v, [r8, lab], new_s, mask=valid)
            plsc.store_scatter(n_v, [r8, lab], new_n, mask=valid)
            plsc.store_scatter(bl_v, [r8, lab], new_bl, mask=valid)
            return carry

        lax.fori_loop(0, MAX_ARCS, arc_body, 0)

    fwr = plsc.load_gather(fw_v, [r8])
    str_ = plsc.load_gather(st_v, [r8])
    plsc.store_scatter(s_v, [r8, zeros16], fwr, mask=lane_lo)
    plsc.store_scatter(n_v, [r8, zeros16], str_, mask=lane_lo)

    pltpu.sync_copy(s_v, out_s_h.at[pl.ds(base, ROWS)])
    pltpu.sync_copy(n_v, out_n_h.at[pl.ds(base, ROWS)])


_advance = functools.partial(
    pl.kernel,
    out_type=(jax.ShapeDtypeStruct((B, VOCAB), jnp.float32),
              jax.ShapeDtypeStruct((B, VOCAB), jnp.int32)),
    mesh=plsc.VectorSubcoreMesh(core_axis_name="c", subcore_axis_name="s"),
    compiler_params=pltpu.CompilerParams(needs_layout_passes=False),
    scratch_types=[
        pltpu.VMEM((L,), jnp.int32),
        pltpu.VMEM((L,), jnp.float32),
        pltpu.VMEM((L,), jnp.float32),
        pltpu.VMEM((L,), jnp.int32),
        pltpu.VMEM((L,), jnp.float32),
        pltpu.VMEM((L,), jnp.int32),
        pltpu.VMEM((L,), jnp.int32),
        pltpu.VMEM((L,), jnp.int32),
        pltpu.VMEM((L,), jnp.int32),
        pltpu.VMEM((L,), jnp.int32),
        pltpu.VMEM((L,), jnp.int32),
        pltpu.VMEM((L,), jnp.float32),
        pltpu.VMEM((L,), jnp.float32),
        pltpu.VMEM((L,), jnp.float32),
        pltpu.VMEM((24, 128), jnp.int32),
        pltpu.VMEM((1024,), jnp.int32),
        pltpu.VMEM((1024,), jnp.float32),
        pltpu.VMEM((1024,), jnp.int32),
        pltpu.VMEM((1024,), jnp.int32),
        pltpu.VMEM((1024,), jnp.float32),
        pltpu.VMEM((1024,), jnp.int32),
        pltpu.VMEM((1024,), jnp.int32),
        pltpu.VMEM((1024,), jnp.float32),
        pltpu.VMEM((1024,), jnp.int32),
        pltpu.VMEM((ROWS, VOCAB), jnp.int32),
        pltpu.VMEM((ROWS, VOCAB), jnp.float32),
        pltpu.VMEM((ROWS, VOCAB), jnp.int32),
        pltpu.SemaphoreType.DMA,
        pltpu.SemaphoreType.DMA,
        pltpu.SemaphoreType.DMA,
        pltpu.SemaphoreType.DMA,
    ],
)(_advance_body)


def kernel(states, model_ids, arcs_weights, ilabels, to_states,
           start_end_arcs, backoff_to_states, backoff_weights,
           final_weights, model2alpha):
    mode = "promise_in_bounds"
    cur0 = states
    cur1 = backoff_to_states.at[cur0].get(mode=mode)
    cur2 = backoff_to_states.at[cur1].get(mode=mode)
    cur3 = backoff_to_states.at[cur2].get(mode=mode)
    curcat = jnp.concatenate([cur0, cur1, cur2])
    se = start_end_arcs.at[curcat].get(mode=mode)
    starts = se[:, 0]
    ends = se[:, 1]
    bwl = backoff_weights.at[curcat].get(mode=mode)
    alpha = model2alpha.at[model_ids].get(mode=mode)
    fwb = final_weights.at[states].get(mode=mode) * alpha
    return _advance(states, alpha, fwb, cur3, starts, ends, bwl,
                    arcs_weights, ilabels, to_states)


# speedup vs baseline: 9.7526x; 1.2712x over previous
import functools

import jax
import jax.numpy as jnp
from jax import lax
from jax.experimental import pallas as pl
from jax.experimental.pallas import tpu as pltpu
from jax.experimental.pallas import tpu_sc as plsc

VOCAB = 1024
B = 256
MAX_ARCS = 64
MAX_ORDER = 3
NUM_WORKERS = 32
ROWS = B // NUM_WORKERS
L = 16


def _advance_body(states_h, mids_h, curcat_h, starts_h, ends_h,
                  bts_h, bwt_h, fw_h, m2a_h, aw_h, il_h, ts_h,
                  out_s_h, out_n_h,
                  st_v, mi_v, cc0_v, cc1_v, cc2_v,
                  alpha_v, fw_v, cur3_v, sf_v,
                  ss0_v, ss1_v, ss2_v, ee0_v, ee1_v, ee2_v,
                  bw0_v, bw1_v, bw2_v, idx_v,
                  lab0_v, w0_v, to0_v, lab1_v, w1_v, to1_v,
                  lab2_v, w2_v, to2_v,
                  bl_v, s_v, n_v,
                  sem_pre, sem_misc, sem_a0, sem_a1, sem_a2):
    cc_ = lax.axis_index("c")
    ss_ = lax.axis_index("s")
    wid = ss_ * 2 + cc_
    base = wid * ROWS

    iota = lax.iota(jnp.int32, L)
    lane_lo = iota < ROWS
    r8 = jnp.bitwise_and(iota, ROWS - 1)
    three16 = jnp.full((L,), MAX_ORDER, jnp.int32)
    zeros16 = jnp.zeros((L,), jnp.int32)

    descs = []
    stage = ((states_h, st_v, 0), (mids_h, mi_v, 0),
             (curcat_h, cc0_v, 0), (curcat_h, cc1_v, B),
             (curcat_h, cc2_v, 2 * B))
    for src, dst, off in stage:
        descs.append(pltpu.async_copy(src.at[pl.ds(off + base, ROWS)],
                                      dst.at[pl.ds(0, ROWS)], sem_pre))
    lvl_refs = ((ss0_v, ee0_v, bw0_v), (ss1_v, ee1_v, bw1_v),
                (ss2_v, ee2_v, bw2_v))
    for lvl in range(MAX_ORDER):
        off = lvl * B + base
        for src, dst in ((starts_h, lvl_refs[lvl][0]),
                         (ends_h, lvl_refs[lvl][1])):
            descs.append(pltpu.async_copy(src.at[pl.ds(off, ROWS)],
                                          dst.at[pl.ds(0, ROWS)], sem_pre))
    for d in descs:
        d.wait()

    sane = lambda v_ref: jnp.where(lane_lo, v_ref[...], 0)
    st_s = sane(st_v)
    mi_s = sane(mi_v)
    curl = [sane(cc0_v), sane(cc1_v), sane(cc2_v)]
    mdescs = [
        pltpu.async_copy(m2a_h.at[mi_s], alpha_v, sem_misc),
        pltpu.async_copy(fw_h.at[st_s], fw_v, sem_misc),
        pltpu.async_copy(bts_h.at[curl[2]], cur3_v, sem_misc),
        pltpu.async_copy(bwt_h.at[curl[0]], bw0_v, sem_misc),
        pltpu.async_copy(bwt_h.at[curl[1]], bw1_v, sem_misc),
        pltpu.async_copy(bwt_h.at[curl[2]], bw2_v, sem_misc),
    ]

    lensv = []
    for lvl in range(MAX_ORDER):
        ss_ref, ee_ref, _ = lvl_refs[lvl]
        stv = plsc.load_gather(ss_ref, [r8])
        env = plsc.load_gather(ee_ref, [r8])
        lensv.append(env - stv)
        for m in range(MAX_ARCS):
            idx_v[lvl * 8 + m // 8, pl.ds((m % 8) * L, L)] = stv + m

    arc_bufs = ((lab0_v, w0_v, to0_v), (lab1_v, w1_v, to1_v),
                (lab2_v, w2_v, to2_v))
    arc_sems = (sem_a0, sem_a1, sem_a2)
    lvl_descs = []
    for lvl in range(MAX_ORDER):
        lab_v, w_v, to_v = arc_bufs[lvl]
        dd = []
        for c4 in range(MAX_ARCS * L // 128):
            dst = pl.ds(128 * c4, 128)
            src = idx_v.at[lvl * 8 + c4]
            dd.append(pltpu.async_copy(il_h.at[src], lab_v.at[dst],
                                       arc_sems[lvl]))
            dd.append(pltpu.async_copy(aw_h.at[src], w_v.at[dst],
                                       arc_sems[lvl]))
            dd.append(pltpu.async_copy(ts_h.at[src], to_v.at[dst],
                                       arc_sems[lvl]))
        lvl_descs.append(dd)

    for d in mdescs:
        d.wait()
    alphav = plsc.load_gather(alpha_v, [r8])
    bwv = [plsc.load_gather(bw_v, [r8]) for _, _, bw_v in lvl_refs]
    ab_lvl = [jnp.zeros((L,), jnp.float32)]
    for lvl in range(MAX_ORDER):
        ab_lvl.append(ab_lvl[lvl] + bwv[lvl])
    sf_v[...] = ab_lvl[MAX_ORDER] * alphav

    def init_row(r, carry):
        rsp = jnp.broadcast_to(r, (L,))
        sfr = plsc.load_gather(sf_v, [rsp])
        c3r = plsc.load_gather(cur3_v, [rsp])

        def init_chunk(ci, carry2, rsp=rsp, sfr=sfr, c3r=c3r):
            colv = ci * L + iota
            plsc.store_scatter(bl_v, [rsp, colv], three16)
            plsc.store_scatter(s_v, [rsp, colv], sfr)
            plsc.store_scatter(n_v, [rsp, colv], c3r)
            return carry2

        lax.fori_loop(0, VOCAB // L, init_chunk, 0)
        return carry

    lax.fori_loop(0, ROWS, init_row, 0)

    for lvl in range(MAX_ORDER):
        for d in lvl_descs[lvl]:
            d.wait()
        lab_v, w_v, to_v = arc_bufs[lvl]
        lvec = jnp.full((L,), lvl, jnp.int32)
        abl = ab_lvl[lvl]
        lens = lensv[lvl]

        def arc_body(j, carry, lab_v=lab_v, w_v=w_v, to_v=to_v,
                     lvec=lvec, abl=abl, lens=lens):
            off = j * L
            lab = jnp.bitwise_and(lab_v[pl.ds(off, L)], VOCAB - 1)
            wv = w_v[pl.ds(off, L)]
            tv = to_v[pl.ds(off, L)]
            valid = lane_lo & (j < lens)
            cur_bl = plsc.load_gather(bl_v, [r8, lab], mask=valid)
            cur_s = plsc.load_gather(s_v, [r8, lab], mask=valid)
            cur_n = plsc.load_gather(n_v, [r8, lab], mask=valid)
            cand = (abl + wv) * alphav
            better = lvec < cur_bl
            eq = lvec == cur_bl
            new_s = jnp.where(better, cand,
                              jnp.where(eq, jnp.maximum(cur_s, cand), cur_s))
            new_n = jnp.where(better, tv,
                              jnp.where(eq, jnp.maximum(cur_n, tv), cur_n))
            new_bl = jnp.minimum(cur_bl, lvec)
            plsc.store_scatter(s_v, [r8, lab], new_s, mask=valid)
            plsc.store_scatter(n_v, [r8, lab], new_n, mask=valid)
            plsc.store_scatter(bl_v, [r8, lab], new_bl, mask=valid)
            return carry

        lax.fori_loop(0, MAX_ARCS, arc_body, 0)

    fwr = plsc.load_gather(fw_v, [r8]) * alphav
    str_ = plsc.load_gather(st_v, [r8])
    plsc.store_scatter(s_v, [r8, zeros16], fwr, mask=lane_lo)
    plsc.store_scatter(n_v, [r8, zeros16], str_, mask=lane_lo)

    pltpu.sync_copy(s_v, out_s_h.at[pl.ds(base, ROWS)])
    pltpu.sync_copy(n_v, out_n_h.at[pl.ds(base, ROWS)])


_advance = functools.partial(
    pl.kernel,
    out_type=(jax.ShapeDtypeStruct((B, VOCAB), jnp.float32),
              jax.ShapeDtypeStruct((B, VOCAB), jnp.int32)),
    mesh=plsc.VectorSubcoreMesh(core_axis_name="c", subcore_axis_name="s"),
    compiler_params=pltpu.CompilerParams(needs_layout_passes=False),
    scratch_types=[
        pltpu.VMEM((L,), jnp.int32),
        pltpu.VMEM((L,), jnp.int32),
        pltpu.VMEM((L,), jnp.int32),
        pltpu.VMEM((L,), jnp.int32),
        pltpu.VMEM((L,), jnp.int32),
        pltpu.VMEM((L,), jnp.float32),
        pltpu.VMEM((L,), jnp.float32),
        pltpu.VMEM((L,), jnp.int32),
        pltpu.VMEM((L,), jnp.float32),
        pltpu.VMEM((L,), jnp.int32),
        pltpu.VMEM((L,), jnp.int32),
        pltpu.VMEM((L,), jnp.int32),
        pltpu.VMEM((L,), jnp.int32),
        pltpu.VMEM((L,), jnp.int32),
        pltpu.VMEM((L,), jnp.int32),
        pltpu.VMEM((L,), jnp.float32),
        pltpu.VMEM((L,), jnp.float32),
        pltpu.VMEM((L,), jnp.float32),
        pltpu.VMEM((24, 128), jnp.int32),
        pltpu.VMEM((1024,), jnp.int32),
        pltpu.VMEM((1024,), jnp.float32),
        pltpu.VMEM((1024,), jnp.int32),
        pltpu.VMEM((1024,), jnp.int32),
        pltpu.VMEM((1024,), jnp.float32),
        pltpu.VMEM((1024,), jnp.int32),
        pltpu.VMEM((1024,), jnp.int32),
        pltpu.VMEM((1024,), jnp.float32),
        pltpu.VMEM((1024,), jnp.int32),
        pltpu.VMEM((ROWS, VOCAB), jnp.int32),
        pltpu.VMEM((ROWS, VOCAB), jnp.float32),
        pltpu.VMEM((ROWS, VOCAB), jnp.int32),
        pltpu.SemaphoreType.DMA,
        pltpu.SemaphoreType.DMA,
        pltpu.SemaphoreType.DMA,
        pltpu.SemaphoreType.DMA,
        pltpu.SemaphoreType.DMA,
    ],
)(_advance_body)


def kernel(states, model_ids, arcs_weights, ilabels, to_states,
           start_end_arcs, backoff_to_states, backoff_weights,
           final_weights, model2alpha):
    mode = "promise_in_bounds"
    cur0 = states
    cur1 = backoff_to_states.at[cur0].get(mode=mode)
    cur2 = backoff_to_states.at[cur1].get(mode=mode)
    curcat = jnp.concatenate([cur0, cur1, cur2])
    se = start_end_arcs.at[curcat].get(mode=mode)
    starts = se[:, 0]
    ends = se[:, 1]
    return _advance(states, model_ids, curcat, starts, ends,
                    backoff_to_states, backoff_weights, final_weights,
                    model2alpha, arcs_weights, ilabels, to_states)


# speedup vs baseline: 9.8318x; 1.0081x over previous
import functools

import jax
import jax.numpy as jnp
from jax import lax
from jax.experimental import pallas as pl
from jax.experimental.pallas import tpu as pltpu
from jax.experimental.pallas import tpu_sc as plsc

VOCAB = 1024
B = 256
MAX_ARCS = 64
MAX_ORDER = 3
NUM_WORKERS = 32
ROWS = B // NUM_WORKERS
L = 16


def _advance_body(states_h, mids_h, curcat_h, starts_h, ends_h,
                  bts_h, bwt_h, fw_h, m2a_h, aw_h, il_h, ts_h,
                  out_s_h, out_n_h,
                  st_v, mi_v, cc0_v, cc1_v, cc2_v,
                  alpha_v, fw_v, cur3_v, sf_v,
                  ss0_v, ss1_v, ss2_v, ee0_v, ee1_v, ee2_v,
                  bw0_v, bw1_v, bw2_v, idx_v,
                  lab0_v, w0_v, to0_v, lab1_v, w1_v, to1_v,
                  lab2_v, w2_v, to2_v,
                  bl_v, s_v, n_v,
                  sem_pre, sem_misc, sem_a0, sem_a1, sem_a2):
    cc_ = lax.axis_index("c")
    ss_ = lax.axis_index("s")
    wid = ss_ * 2 + cc_
    base = wid * ROWS

    iota = lax.iota(jnp.int32, L)
    lane_lo = iota < ROWS
    r8 = jnp.bitwise_and(iota, ROWS - 1)
    three16 = jnp.full((L,), MAX_ORDER, jnp.int32)
    zeros16 = jnp.zeros((L,), jnp.int32)

    descs = []
    stage = ((states_h, st_v, 0), (mids_h, mi_v, 0),
             (curcat_h, cc0_v, 0), (curcat_h, cc1_v, B),
             (curcat_h, cc2_v, 2 * B))
    for src, dst, off in stage:
        descs.append(pltpu.async_copy(src.at[pl.ds(off + base, ROWS)],
                                      dst.at[pl.ds(0, ROWS)], sem_pre))
    lvl_refs = ((ss0_v, ee0_v, bw0_v), (ss1_v, ee1_v, bw1_v),
                (ss2_v, ee2_v, bw2_v))
    for lvl in range(MAX_ORDER):
        off = lvl * B + base
        for src, dst in ((starts_h, lvl_refs[lvl][0]),
                         (ends_h, lvl_refs[lvl][1])):
            descs.append(pltpu.async_copy(src.at[pl.ds(off, ROWS)],
                                          dst.at[pl.ds(0, ROWS)], sem_pre))
    for d in descs:
        d.wait()

    sane = lambda v_ref: jnp.where(lane_lo, v_ref[...], 0)
    st_s = sane(st_v)
    mi_s = sane(mi_v)
    curl = [sane(cc0_v), sane(cc1_v), sane(cc2_v)]
    mdescs = [
        pltpu.async_copy(m2a_h.at[mi_s], alpha_v, sem_misc),
        pltpu.async_copy(fw_h.at[st_s], fw_v, sem_misc),
        pltpu.async_copy(bts_h.at[curl[2]], cur3_v, sem_misc),
        pltpu.async_copy(bwt_h.at[curl[0]], bw0_v, sem_misc),
        pltpu.async_copy(bwt_h.at[curl[1]], bw1_v, sem_misc),
        pltpu.async_copy(bwt_h.at[curl[2]], bw2_v, sem_misc),
    ]

    lensv = []
    for lvl in range(MAX_ORDER):
        ss_ref, ee_ref, _ = lvl_refs[lvl]
        stv = plsc.load_gather(ss_ref, [r8])
        env = plsc.load_gather(ee_ref, [r8])
        lensv.append(env - stv)
        for m in range(MAX_ARCS):
            idx_v[lvl * 8 + m // 8, pl.ds((m % 8) * L, L)] = stv + m

    arc_bufs = ((lab0_v, w0_v, to0_v), (lab1_v, w1_v, to1_v),
                (lab2_v, w2_v, to2_v))
    arc_sems = (sem_a0, sem_a1, sem_a2)
    lvl_descs = []
    for lvl in range(MAX_ORDER):
        lab_v, w_v, to_v = arc_bufs[lvl]
        dd = []
        for c4 in range(MAX_ARCS * L // 128):
            dst = pl.ds(128 * c4, 128)
            src = idx_v.at[lvl * 8 + c4]
            dd.append(pltpu.async_copy(il_h.at[src], lab_v.at[dst],
                                       arc_sems[lvl]))
            dd.append(pltpu.async_copy(aw_h.at[src], w_v.at[dst],
                                       arc_sems[lvl]))
            dd.append(pltpu.async_copy(ts_h.at[src], to_v.at[dst],
                                       arc_sems[lvl]))
        lvl_descs.append(dd)

    for d in mdescs:
        d.wait()
    alphav = plsc.load_gather(alpha_v, [r8])
    bwv = [plsc.load_gather(bw_v, [r8]) for _, _, bw_v in lvl_refs]
    ab_lvl = [jnp.zeros((L,), jnp.float32)]
    for lvl in range(MAX_ORDER):
        ab_lvl.append(ab_lvl[lvl] + bwv[lvl])
    sf_v[...] = ab_lvl[MAX_ORDER] * alphav

    def init_row(r, carry):
        rsp = jnp.broadcast_to(r, (L,))
        sfr = plsc.load_gather(sf_v, [rsp])
        c3r = plsc.load_gather(cur3_v, [rsp])

        def init_chunk(ci, carry2, rsp=rsp, sfr=sfr, c3r=c3r):
            for half in range(2):
                colv = (2 * ci + half) * L + iota
                plsc.store_scatter(bl_v, [rsp, colv], three16)
                plsc.store_scatter(s_v, [rsp, colv], sfr)
                plsc.store_scatter(n_v, [rsp, colv], c3r)
            return carry2

        lax.fori_loop(0, VOCAB // (2 * L), init_chunk, 0)
        return carry

    lax.fori_loop(0, ROWS, init_row, 0)

    for lvl in range(MAX_ORDER):
        for d in lvl_descs[lvl]:
            d.wait()
        lab_v, w_v, to_v = arc_bufs[lvl]
        lvec = jnp.full((L,), lvl, jnp.int32)
        abl = ab_lvl[lvl]
        lens = lensv[lvl]

        def arc_body(j, carry, lab_v=lab_v, w_v=w_v, to_v=to_v,
                     lvec=lvec, abl=abl, lens=lens):
            off = j * L
            lab = lab_v[pl.ds(off, L)]
            wv = w_v[pl.ds(off, L)]
            tv = to_v[pl.ds(off, L)]
            valid = lane_lo & (j < lens)
            cur_bl = plsc.load_gather(bl_v, [r8, lab], mask=valid)
            cur_s = plsc.load_gather(s_v, [r8, lab], mask=valid)
            cur_n = plsc.load_gather(n_v, [r8, lab], mask=valid)
            cand = (abl + wv) * alphav
            better = lvec < cur_bl
            eq = lvec == cur_bl
            new_s = jnp.where(better, cand,
                              jnp.where(eq, jnp.maximum(cur_s, cand), cur_s))
            new_n = jnp.where(better, tv,
                              jnp.where(eq, jnp.maximum(cur_n, tv), cur_n))
            new_bl = jnp.minimum(cur_bl, lvec)
            plsc.store_scatter(s_v, [r8, lab], new_s, mask=valid)
            plsc.store_scatter(n_v, [r8, lab], new_n, mask=valid)
            plsc.store_scatter(bl_v, [r8, lab], new_bl, mask=valid)
            return carry

        maxlen = lax.reduce_max(jnp.where(lane_lo, lens, 0), (0,))
        lax.fori_loop(0, maxlen, arc_body, 0)

    fwr = plsc.load_gather(fw_v, [r8]) * alphav
    str_ = plsc.load_gather(st_v, [r8])
    plsc.store_scatter(s_v, [r8, zeros16], fwr, mask=lane_lo)
    plsc.store_scatter(n_v, [r8, zeros16], str_, mask=lane_lo)

    pltpu.sync_copy(s_v, out_s_h.at[pl.ds(base, ROWS)])
    pltpu.sync_copy(n_v, out_n_h.at[pl.ds(base, ROWS)])


_advance = functools.partial(
    pl.kernel,
    out_type=(jax.ShapeDtypeStruct((B, VOCAB), jnp.float32),
              jax.ShapeDtypeStruct((B, VOCAB), jnp.int32)),
    mesh=plsc.VectorSubcoreMesh(core_axis_name="c", subcore_axis_name="s"),
    compiler_params=pltpu.CompilerParams(needs_layout_passes=False),
    scratch_types=[
        pltpu.VMEM((L,), jnp.int32),
        pltpu.VMEM((L,), jnp.int32),
        pltpu.VMEM((L,), jnp.int32),
        pltpu.VMEM((L,), jnp.int32),
        pltpu.VMEM((L,), jnp.int32),
        pltpu.VMEM((L,), jnp.float32),
        pltpu.VMEM((L,), jnp.float32),
        pltpu.VMEM((L,), jnp.int32),
        pltpu.VMEM((L,), jnp.float32),
        pltpu.VMEM((L,), jnp.int32),
        pltpu.VMEM((L,), jnp.int32),
        pltpu.VMEM((L,), jnp.int32),
        pltpu.VMEM((L,), jnp.int32),
        pltpu.VMEM((L,), jnp.int32),
        pltpu.VMEM((L,), jnp.int32),
        pltpu.VMEM((L,), jnp.float32),
        pltpu.VMEM((L,), jnp.float32),
        pltpu.VMEM((L,), jnp.float32),
        pltpu.VMEM((24, 128), jnp.int32),
        pltpu.VMEM((1024,), jnp.int32),
        pltpu.VMEM((1024,), jnp.float32),
        pltpu.VMEM((1024,), jnp.int32),
        pltpu.VMEM((1024,), jnp.int32),
        pltpu.VMEM((1024,), jnp.float32),
        pltpu.VMEM((1024,), jnp.int32),
        pltpu.VMEM((1024,), jnp.int32),
        pltpu.VMEM((1024,), jnp.float32),
        pltpu.VMEM((1024,), jnp.int32),
        pltpu.VMEM((ROWS, VOCAB), jnp.int32),
        pltpu.VMEM((ROWS, VOCAB), jnp.float32),
        pltpu.VMEM((ROWS, VOCAB), jnp.int32),
        pltpu.SemaphoreType.DMA,
        pltpu.SemaphoreType.DMA,
        pltpu.SemaphoreType.DMA,
        pltpu.SemaphoreType.DMA,
        pltpu.SemaphoreType.DMA,
    ],
)(_advance_body)


def kernel(states, model_ids, arcs_weights, ilabels, to_states,
           start_end_arcs, backoff_to_states, backoff_weights,
           final_weights, model2alpha):
    mode = "promise_in_bounds"
    cur0 = states
    cur1 = backoff_to_states.at[cur0].get(mode=mode)
    cur2 = backoff_to_states.at[cur1].get(mode=mode)
    curcat = jnp.concatenate([cur0, cur1, cur2])
    se = start_end_arcs.at[curcat].get(mode=mode)
    starts = se[:, 0]
    ends = se[:, 1]
    return _advance(states, model_ids, curcat, starts, ends,
                    backoff_to_states, backoff_weights, final_weights,
                    model2alpha, arcs_weights, ilabels, to_states)
